# baseline (speedup 1.0000x reference)
import jax
import jax.numpy as jnp
from jax import lax
from jax.experimental import pallas as pl
from jax.experimental.pallas import tpu as pltpu
from jax.experimental.pallas import tpu_sc as plsc

ROW = 10000
MAIN = 9984
TAIL = ROW - MAIN
TPAD = 128
BATCH = 8192
SC_N = 3584
TC_N = BATCH - SC_N
NC, NS = 2, 16
NW = NC * NS
PER_W = SC_N // NW
K = 2
NGR = PER_W // K
LANES = 16
CHUNKS = MAIN // LANES
U = 24
NJ = CHUNKS // U
NACC = 6
TCHUNKS = TPAD // LANES
GPS = LANES // K
NSG = PER_W // LANES

BP = 256
NB = TC_N // BP


def _pair_dot(rm1, rt1, rm2, rt2, p):
    def body(j, accs):
        base = j * (U * LANES)
        accs = list(accs)
        for u in range(U):
            x = rm1[p, pl.ds(base + u * LANES, LANES)]
            y = rm2[p, pl.ds(base + u * LANES, LANES)]
            accs[u % NACC] = accs[u % NACC] + x * y
        return tuple(accs)

    accs = tuple(jnp.zeros((LANES,), jnp.float32) for _ in range(NACC))
    accs = lax.fori_loop(0, NJ, body, accs)
    accs = list(accs)
    for t in range(TCHUNKS):
        x = rt1[p, pl.ds(t * LANES, LANES)]
        y = rt2[p, pl.ds(t * LANES, LANES)]
        accs[t % NACC] = accs[t % NACC] + x * y
    tot = accs[0]
    for u in range(1, NACC):
        tot = tot + accs[u]
    lane = lax.broadcasted_iota(jnp.int32, (LANES,), 0)
    for sh in (8, 4, 2, 1):
        idx = jnp.bitwise_and(lane + sh, LANES - 1)
        tot = tot + tot.at[idx].get(mode="promise_in_bounds",
                                    unique_indices=True)
    return tot


def _sc_body(a_hbm, atail_hbm, n1_hbm, n2_hbm, out_hbm,
             idx1_v, idx2_v,
             m1a, m1b, m2a, m2b, t1a, t1b, t2a, t2b, out_v,
             s1a, s1b, s2a, s2b):
    wid = lax.axis_index("s") * NC + lax.axis_index("c")
    pltpu.sync_copy(n1_hbm.at[pl.ds(wid * NGR, NGR)], idx1_v)
    pltpu.sync_copy(n2_hbm.at[pl.ds(wid * NGR, NGR)], idx2_v)
    lane = lax.broadcasted_iota(jnp.int32, (LANES,), 0)

    a_main = a_hbm.at[:, pl.ds(0, MAIN)]
    m1 = (m1a, m1b)
    m2 = (m2a, m2b)
    t1 = (t1a, t1b)
    t2 = (t2a, t2b)
    sem1 = (s1a, s1b)
    sem2 = (s2a, s2b)

    def fire(g, par):
        pltpu.async_copy(a_main.at[idx1_v.at[g]], m1[par], sem1[par])
        pltpu.async_copy(atail_hbm.at[idx1_v.at[g]], t1[par], sem1[par])
        pltpu.async_copy(a_main.at[idx2_v.at[g]], m2[par], sem2[par])
        pltpu.async_copy(atail_hbm.at[idx2_v.at[g]], t2[par], sem2[par])

    def drain(par):
        pltpu.make_async_copy(a_main.at[idx1_v.at[0]], m1[par], sem1[par]).wait()
        pltpu.make_async_copy(atail_hbm.at[idx1_v.at[0]], t1[par], sem1[par]).wait()
        pltpu.make_async_copy(a_main.at[idx2_v.at[0]], m2[par], sem2[par]).wait()
        pltpu.make_async_copy(atail_hbm.at[idx2_v.at[0]], t2[par], sem2[par]).wait()

    fire(0, 0)

    def supergroup(sg, carry):
        vec = jnp.zeros((LANES,), jnp.float32)
        for q in range(GPS):
            par = q % 2
            g = sg * GPS + q
            drain(par)
            g_next = g + 1

            @pl.when(g_next < NGR)
            def _():
                fire(g_next, (q + 1) % 2)

            for p in range(K):
                s = _pair_dot(m1[par], t1[par], m2[par], t2[par], p)
                vec = jnp.where(lane == (q * K + p), s, vec)
        out_v[pl.ds(sg * LANES, LANES)] = vec
        return carry

    lax.fori_loop(0, NSG, supergroup, 0)
    pltpu.sync_copy(out_v, out_hbm.at[pl.ds(wid * PER_W, PER_W)])


def _tc_body(n1_sref, n2_sref, a_ref, out_ref, rows1, rows2, sems):
    i = pl.program_id(0)

    def fire(step, par):
        for p in range(BP):
            i1 = n1_sref[step * BP + p]
            i2 = n2_sref[step * BP + p]
            pltpu.make_async_copy(
                a_ref.at[pl.ds(i1, 1)], rows1.at[par, pl.ds(p, 1)],
                sems.at[par]).start()
            pltpu.make_async_copy(
                a_ref.at[pl.ds(i2, 1)], rows2.at[par, pl.ds(p, 1)],
                sems.at[par]).start()

    def drain(step, par):
        pltpu.make_async_copy(
            a_ref.at[pl.ds(0, BP)], rows1.at[par], sems.at[par]).wait()
        pltpu.make_async_copy(
            a_ref.at[pl.ds(0, BP)], rows2.at[par], sems.at[par]).wait()

    @pl.when(i == 0)
    def _():
        fire(0, 0)

    par = lax.rem(i, 2)

    @pl.when(i + 1 < NB)
    def _():
        fire(i + 1, lax.rem(i + 1, 2))

    drain(i, par)
    r1 = rows1[par]
    r2 = rows2[par]
    out_ref[0, 0, :] = jnp.sum(r1 * r2, axis=1)


def kernel(A, nodes1, nodes2):
    n1 = nodes1.astype(jnp.int32)
    n2 = nodes2.astype(jnp.int32)
    n1_sc = n1[:SC_N].reshape(SC_N // K, K)
    n2_sc = n2[:SC_N].reshape(SC_N // K, K)
    a_tail = jnp.pad(A[:, MAIN:], ((0, 0), (0, TPAD - TAIL)))
    mesh = plsc.VectorSubcoreMesh(core_axis_name="c", subcore_axis_name="s")
    sc_fn = pl.kernel(
        _sc_body,
        out_type=jax.ShapeDtypeStruct((SC_N,), jnp.float32),
        mesh=mesh,
        compiler_params=pltpu.CompilerParams(use_tc_tiling_on_sc=True),
        scratch_types=[
            pltpu.VMEM((NGR, K), jnp.int32),
            pltpu.VMEM((NGR, K), jnp.int32),
            pltpu.VMEM((K, MAIN), jnp.float32),
            pltpu.VMEM((K, MAIN), jnp.float32),
            pltpu.VMEM((K, MAIN), jnp.float32),
            pltpu.VMEM((K, MAIN), jnp.float32),
            pltpu.VMEM((K, TPAD), jnp.float32),
            pltpu.VMEM((K, TPAD), jnp.float32),
            pltpu.VMEM((K, TPAD), jnp.float32),
            pltpu.VMEM((K, TPAD), jnp.float32),
            pltpu.VMEM((PER_W,), jnp.float32),
            pltpu.SemaphoreType.DMA,
            pltpu.SemaphoreType.DMA,
            pltpu.SemaphoreType.DMA,
            pltpu.SemaphoreType.DMA,
        ],
    )
    sc_out = sc_fn(A, a_tail, n1_sc, n2_sc)

    tc_fn = pl.pallas_call(
        _tc_body,
        grid_spec=pltpu.PrefetchScalarGridSpec(
            num_scalar_prefetch=2,
            grid=(NB,),
            in_specs=[pl.BlockSpec(memory_space=pl.ANY)],
            out_specs=pl.BlockSpec((1, 1, BP), lambda i, n1, n2: (i, 0, 0)),
            scratch_shapes=[
                pltpu.VMEM((2, BP, ROW), jnp.float32),
                pltpu.VMEM((2, BP, ROW), jnp.float32),
                pltpu.SemaphoreType.DMA((2,)),
            ],
        ),
        out_shape=jax.ShapeDtypeStruct((NB, 1, BP), jnp.float32),
    )
    tc_out = tc_fn(n1[SC_N:], n2[SC_N:], A).reshape(TC_N)
    return jnp.concatenate([sc_out, tc_out])


# speedup vs baseline: 1.0020x; 1.0020x over previous
import jax
import jax.numpy as jnp
from jax import lax
from jax.experimental import pallas as pl
from jax.experimental.pallas import tpu as pltpu
from jax.experimental.pallas import tpu_sc as plsc

ROW = 10000
MAIN = 9984
TAIL = ROW - MAIN
TPAD = 128
BATCH = 8192
SC_N = 3584
TC_N = BATCH - SC_N
NC, NS = 2, 16
NW = NC * NS
PER_W = SC_N // NW
K = 2
NGR = PER_W // K
LANES = 16
CHUNKS = MAIN // LANES
U = 24
NJ = CHUNKS // U
NACC = 6
TCHUNKS = TPAD // LANES
GPS = LANES // K
NSG = PER_W // LANES

BP = 128
NB = TC_N // BP
NQ = 4


def _pair_dot(rm1, rt1, rm2, rt2, p):
    def body(j, accs):
        base = j * (U * LANES)
        accs = list(accs)
        for u in range(U):
            x = rm1[p, pl.ds(base + u * LANES, LANES)]
            y = rm2[p, pl.ds(base + u * LANES, LANES)]
            accs[u % NACC] = accs[u % NACC] + x * y
        return tuple(accs)

    accs = tuple(jnp.zeros((LANES,), jnp.float32) for _ in range(NACC))
    accs = lax.fori_loop(0, NJ, body, accs)
    accs = list(accs)
    for t in range(TCHUNKS):
        x = rt1[p, pl.ds(t * LANES, LANES)]
        y = rt2[p, pl.ds(t * LANES, LANES)]
        accs[t % NACC] = accs[t % NACC] + x * y
    tot = accs[0]
    for u in range(1, NACC):
        tot = tot + accs[u]
    lane = lax.broadcasted_iota(jnp.int32, (LANES,), 0)
    for sh in (8, 4, 2, 1):
        idx = jnp.bitwise_and(lane + sh, LANES - 1)
        tot = tot + tot.at[idx].get(mode="promise_in_bounds",
                                    unique_indices=True)
    return tot


def _sc_body(a_hbm, atail_hbm, n1_hbm, n2_hbm, out_hbm,
             idx1_v, idx2_v,
             m1a, m1b, m2a, m2b, t1a, t1b, t2a, t2b, out_v,
             s1a, s1b, s2a, s2b):
    wid = lax.axis_index("s") * NC + lax.axis_index("c")
    pltpu.sync_copy(n1_hbm.at[pl.ds(wid * NGR, NGR)], idx1_v)
    pltpu.sync_copy(n2_hbm.at[pl.ds(wid * NGR, NGR)], idx2_v)
    lane = lax.broadcasted_iota(jnp.int32, (LANES,), 0)

    a_main = a_hbm.at[:, pl.ds(0, MAIN)]
    m1 = (m1a, m1b)
    m2 = (m2a, m2b)
    t1 = (t1a, t1b)
    t2 = (t2a, t2b)
    sem1 = (s1a, s1b)
    sem2 = (s2a, s2b)

    def fire(g, par):
        pltpu.async_copy(a_main.at[idx1_v.at[g]], m1[par], sem1[par])
        pltpu.async_copy(atail_hbm.at[idx1_v.at[g]], t1[par], sem1[par])
        pltpu.async_copy(a_main.at[idx2_v.at[g]], m2[par], sem2[par])
        pltpu.async_copy(atail_hbm.at[idx2_v.at[g]], t2[par], sem2[par])

    def drain(par):
        pltpu.make_async_copy(a_main.at[idx1_v.at[0]], m1[par], sem1[par]).wait()
        pltpu.make_async_copy(atail_hbm.at[idx1_v.at[0]], t1[par], sem1[par]).wait()
        pltpu.make_async_copy(a_main.at[idx2_v.at[0]], m2[par], sem2[par]).wait()
        pltpu.make_async_copy(atail_hbm.at[idx2_v.at[0]], t2[par], sem2[par]).wait()

    fire(0, 0)

    def supergroup(sg, carry):
        vec = jnp.zeros((LANES,), jnp.float32)
        for q in range(GPS):
            par = q % 2
            g = sg * GPS + q
            drain(par)
            g_next = g + 1

            @pl.when(g_next < NGR)
            def _():
                fire(g_next, (q + 1) % 2)

            for p in range(K):
                s = _pair_dot(m1[par], t1[par], m2[par], t2[par], p)
                vec = jnp.where(lane == (q * K + p), s, vec)
        out_v[pl.ds(sg * LANES, LANES)] = vec
        return carry

    lax.fori_loop(0, NSG, supergroup, 0)
    pltpu.sync_copy(out_v, out_hbm.at[pl.ds(wid * PER_W, PER_W)])


def _tc_body(n1_sref, n2_sref, a_ref, out_ref, rows1, rows2, sems):
    i = pl.program_id(0)

    def fire(step, par):
        for p in range(BP):
            i1 = n1_sref[step * BP + p]
            i2 = n2_sref[step * BP + p]
            pltpu.make_async_copy(
                a_ref.at[pl.ds(i1, 1)], rows1.at[par, pl.ds(p, 1)],
                sems.at[par, p % NQ]).start()
            pltpu.make_async_copy(
                a_ref.at[pl.ds(i2, 1)], rows2.at[par, pl.ds(p, 1)],
                sems.at[par, p % NQ]).start()

    def drain(step, par):
        for q in range(NQ):
            pltpu.make_async_copy(
                a_ref.at[pl.ds(0, 2 * (BP // NQ))],
                rows1.at[par, pl.ds(0, 2 * (BP // NQ))],
                sems.at[par, q]).wait()

    @pl.when(i == 0)
    def _():
        fire(0, 0)

    par = lax.rem(i, 2)

    @pl.when(i + 1 < NB)
    def _():
        fire(i + 1, lax.rem(i + 1, 2))

    drain(i, par)
    r1 = rows1[par]
    r2 = rows2[par]
    out_ref[0, 0, :] = jnp.sum(r1 * r2, axis=1)


def kernel(A, nodes1, nodes2):
    n1 = nodes1.astype(jnp.int32)
    n2 = nodes2.astype(jnp.int32)
    n1_sc = n1[:SC_N].reshape(SC_N // K, K)
    n2_sc = n2[:SC_N].reshape(SC_N // K, K)
    a_tail = jnp.pad(A[:, MAIN:], ((0, 0), (0, TPAD - TAIL)))
    mesh = plsc.VectorSubcoreMesh(core_axis_name="c", subcore_axis_name="s")
    sc_fn = pl.kernel(
        _sc_body,
        out_type=jax.ShapeDtypeStruct((SC_N,), jnp.float32),
        mesh=mesh,
        compiler_params=pltpu.CompilerParams(use_tc_tiling_on_sc=True),
        scratch_types=[
            pltpu.VMEM((NGR, K), jnp.int32),
            pltpu.VMEM((NGR, K), jnp.int32),
            pltpu.VMEM((K, MAIN), jnp.float32),
            pltpu.VMEM((K, MAIN), jnp.float32),
            pltpu.VMEM((K, MAIN), jnp.float32),
            pltpu.VMEM((K, MAIN), jnp.float32),
            pltpu.VMEM((K, TPAD), jnp.float32),
            pltpu.VMEM((K, TPAD), jnp.float32),
            pltpu.VMEM((K, TPAD), jnp.float32),
            pltpu.VMEM((K, TPAD), jnp.float32),
            pltpu.VMEM((PER_W,), jnp.float32),
            pltpu.SemaphoreType.DMA,
            pltpu.SemaphoreType.DMA,
            pltpu.SemaphoreType.DMA,
            pltpu.SemaphoreType.DMA,
        ],
    )
    sc_out = sc_fn(A, a_tail, n1_sc, n2_sc)

    tc_fn = pl.pallas_call(
        _tc_body,
        grid_spec=pltpu.PrefetchScalarGridSpec(
            num_scalar_prefetch=2,
            grid=(NB,),
            in_specs=[pl.BlockSpec(memory_space=pl.ANY)],
            out_specs=pl.BlockSpec((1, 1, BP), lambda i, n1, n2: (i, 0, 0)),
            scratch_shapes=[
                pltpu.VMEM((2, BP, ROW), jnp.float32),
                pltpu.VMEM((2, BP, ROW), jnp.float32),
                pltpu.SemaphoreType.DMA((2, NQ)),
            ],
        ),
        out_shape=jax.ShapeDtypeStruct((NB, 1, BP), jnp.float32),
    )
    tc_out = tc_fn(n1[SC_N:], n2[SC_N:], A).reshape(TC_N)
    return jnp.concatenate([sc_out, tc_out])
